# baseline (speedup 1.0000x reference)
import functools

import jax
import jax.numpy as jnp
from jax.experimental import pallas as pl
from jax.experimental.pallas import tpu as pltpu

_N_CLASSES = 1000
_CTX_LEN = 77
_N_CTX = 4
_D = 768
_SUF = _CTX_LEN - 1 - _N_CTX

_CB = 20


def _body(prefix_ref, ctx_ref, suffix_ref, out_ref):
    out_ref[:, 0:1, :] = prefix_ref[...]
    out_ref[:, 1 : 1 + _N_CTX, :] = jnp.broadcast_to(
        ctx_ref[...][None], (_CB, _N_CTX, _D)
    )
    out_ref[:, 1 + _N_CTX :, :] = suffix_ref[...]


@jax.jit
def _prompt_concat(token_prefix, ctx_embedding, token_suffix):
    grid = (_N_CLASSES // _CB,)
    return pl.pallas_call(
        _body,
        grid=grid,
        in_specs=[
            pl.BlockSpec((_CB, 1, _D), lambda i: (i, 0, 0)),
            pl.BlockSpec((_N_CTX, _D), lambda i: (0, 0)),
            pl.BlockSpec((_CB, _SUF, _D), lambda i: (i, 0, 0)),
        ],
        out_specs=pl.BlockSpec((_CB, _CTX_LEN, _D), lambda i: (i, 0, 0)),
        out_shape=jax.ShapeDtypeStruct((_N_CLASSES, _CTX_LEN, _D), jnp.float32),
        compiler_params=pltpu.CompilerParams(
            dimension_semantics=("arbitrary",),
        ),
    )(token_prefix, ctx_embedding, token_suffix)


def kernel(token_prefix, ctx_embedding, token_suffix, eos_position):
    prompts = _prompt_concat(token_prefix, ctx_embedding, token_suffix)
    return (prompts, eos_position)


# speedup vs baseline: 1.0052x; 1.0052x over previous
import functools

import jax
import jax.numpy as jnp
from jax.experimental import pallas as pl
from jax.experimental.pallas import tpu as pltpu

_N_CLASSES = 1000
_CTX_LEN = 77
_N_CTX = 4
_D = 768
_SUF = _CTX_LEN - 1 - _N_CTX

_CB = 50


def _body(prefix_ref, ctx_ref, suffix_ref, out_ref):
    out_ref[:, 0:1, :] = prefix_ref[...]
    out_ref[:, 1 : 1 + _N_CTX, :] = jnp.broadcast_to(
        ctx_ref[...][None], (_CB, _N_CTX, _D)
    )
    out_ref[:, 1 + _N_CTX :, :] = suffix_ref[...]


@jax.jit
def _prompt_concat(token_prefix, ctx_embedding, token_suffix):
    grid = (_N_CLASSES // _CB,)
    return pl.pallas_call(
        _body,
        grid=grid,
        in_specs=[
            pl.BlockSpec((_CB, 1, _D), lambda i: (i, 0, 0)),
            pl.BlockSpec((_N_CTX, _D), lambda i: (0, 0)),
            pl.BlockSpec((_CB, _SUF, _D), lambda i: (i, 0, 0)),
        ],
        out_specs=pl.BlockSpec((_CB, _CTX_LEN, _D), lambda i: (i, 0, 0)),
        out_shape=jax.ShapeDtypeStruct((_N_CLASSES, _CTX_LEN, _D), jnp.float32),
        compiler_params=pltpu.CompilerParams(
            dimension_semantics=("arbitrary",),
        ),
    )(token_prefix, ctx_embedding, token_suffix)


def kernel(token_prefix, ctx_embedding, token_suffix, eos_position):
    prompts = _prompt_concat(token_prefix, ctx_embedding, token_suffix)
    return (prompts, eos_position)
